# initial kernel scaffold (unmeasured)
import jax
import jax.numpy as jnp
from jax import lax
from jax.experimental import pallas as pl
from jax.experimental.pallas import tpu as pltpu


def kernel(
    x,
):
    def body(*refs):
        pass

    out_shape = jax.ShapeDtypeStruct(..., jnp.float32)
    return pl.pallas_call(body, out_shape=out_shape)(...)


# baseline (device time: 12239 ns/iter reference)
import jax
import jax.numpy as jnp
from jax import lax
from jax.experimental import pallas as pl
from jax.experimental.pallas import tpu as pltpu

M = 512
N = 1024
HALF = N // 2


def kernel(x):
    x2 = x.reshape(M, N)

    def body(x_ref, out_ref, send_buf, recv_buf, send_sem, recv_sem):
        my_x = lax.axis_index("x")
        my_y = lax.axis_index("y")
        my_z = lax.axis_index("z")
        peer = (my_x, my_y, 1 - my_z)

        other_off = (1 - my_z) * HALF
        send_buf[...] = x_ref[:, pl.ds(other_off, HALF)].astype(jnp.bfloat16)

        barrier_sem = pltpu.get_barrier_semaphore()
        pl.semaphore_signal(
            barrier_sem,
            inc=1,
            device_id=peer,
            device_id_type=pl.DeviceIdType.MESH,
        )
        pl.semaphore_wait(barrier_sem, 1)

        rdma = pltpu.make_async_remote_copy(
            src_ref=send_buf,
            dst_ref=recv_buf,
            send_sem=send_sem,
            recv_sem=recv_sem,
            device_id=peer,
            device_id_type=pl.DeviceIdType.MESH,
        )
        rdma.start()
        rdma.wait()

        my_off = my_z * HALF
        out_ref[...] = x_ref[:, pl.ds(my_off, HALF)] + recv_buf[...].astype(
            jnp.float32
        )

    return pl.pallas_call(
        body,
        out_shape=jax.ShapeDtypeStruct((M, HALF), jnp.float32),
        in_specs=[pl.BlockSpec(memory_space=pltpu.VMEM)],
        out_specs=pl.BlockSpec(memory_space=pltpu.VMEM),
        scratch_shapes=[
            pltpu.VMEM((M, HALF), jnp.bfloat16),
            pltpu.VMEM((M, HALF), jnp.bfloat16),
            pltpu.SemaphoreType.DMA,
            pltpu.SemaphoreType.DMA,
        ],
        compiler_params=pltpu.CompilerParams(collective_id=0),
    )(x2)


# device time: 12156 ns/iter; 1.0068x vs baseline; 1.0068x over previous
import jax
import jax.numpy as jnp
from jax import lax
from jax.experimental import pallas as pl
from jax.experimental.pallas import tpu as pltpu

M = 512
N = 1024
HALF = N // 2


def kernel(x):
    x2 = x.reshape(M, N)

    R0 = M // 2

    def body(x_ref, out_ref, send_buf, recv_buf, send_sems, recv_sems):
        my_x = lax.axis_index("x")
        my_y = lax.axis_index("y")
        my_z = lax.axis_index("z")
        peer = (my_x, my_y, 1 - my_z)

        barrier_sem = pltpu.get_barrier_semaphore()
        pl.semaphore_signal(
            barrier_sem,
            inc=1,
            device_id=peer,
            device_id_type=pl.DeviceIdType.MESH,
        )

        other_off = (1 - my_z) * HALF
        send_buf[0:R0] = x_ref[0:R0, pl.ds(other_off, HALF)].astype(
            jnp.bfloat16
        )
        pl.semaphore_wait(barrier_sem, 1)

        rdma0 = pltpu.make_async_remote_copy(
            src_ref=send_buf.at[0:R0],
            dst_ref=recv_buf.at[0:R0],
            send_sem=send_sems.at[0],
            recv_sem=recv_sems.at[0],
            device_id=peer,
            device_id_type=pl.DeviceIdType.MESH,
        )
        rdma0.start()

        send_buf[R0:M] = x_ref[R0:M, pl.ds(other_off, HALF)].astype(
            jnp.bfloat16
        )
        rdma1 = pltpu.make_async_remote_copy(
            src_ref=send_buf.at[R0:M],
            dst_ref=recv_buf.at[R0:M],
            send_sem=send_sems.at[1],
            recv_sem=recv_sems.at[1],
            device_id=peer,
            device_id_type=pl.DeviceIdType.MESH,
        )
        rdma1.start()

        my_off = my_z * HALF
        out_ref[...] = x_ref[:, pl.ds(my_off, HALF)]

        rdma0.wait()
        out_ref[0:R0] = out_ref[0:R0] + recv_buf[0:R0].astype(jnp.float32)
        rdma1.wait()
        out_ref[R0:M] = out_ref[R0:M] + recv_buf[R0:M].astype(jnp.float32)

    return pl.pallas_call(
        body,
        out_shape=jax.ShapeDtypeStruct((M, HALF), jnp.float32),
        in_specs=[pl.BlockSpec(memory_space=pltpu.VMEM)],
        out_specs=pl.BlockSpec(memory_space=pltpu.VMEM),
        scratch_shapes=[
            pltpu.VMEM((M, HALF), jnp.bfloat16),
            pltpu.VMEM((M, HALF), jnp.bfloat16),
            pltpu.SemaphoreType.DMA((2,)),
            pltpu.SemaphoreType.DMA((2,)),
        ],
        compiler_params=pltpu.CompilerParams(collective_id=0),
    )(x2)


# device time: 12112 ns/iter; 1.0105x vs baseline; 1.0036x over previous
import jax
import jax.numpy as jnp
from jax import lax
from jax.experimental import pallas as pl
from jax.experimental.pallas import tpu as pltpu

M = 512
N = 1024
HALF = N // 2


def kernel(x):
    R0 = M // 2

    def body(x_ref, out_ref, send_buf, recv_buf, send_sems, recv_sems):
        my_x = lax.axis_index("x")
        my_y = lax.axis_index("y")
        my_z = lax.axis_index("z")
        peer = (my_x, my_y, 1 - my_z)

        barrier_sem = pltpu.get_barrier_semaphore()
        pl.semaphore_signal(
            barrier_sem,
            inc=1,
            device_id=peer,
            device_id_type=pl.DeviceIdType.MESH,
        )

        other_off = (1 - my_z) * HALF
        send_buf[0:R0] = x_ref[0, 0:R0, pl.ds(other_off, HALF)].astype(
            jnp.bfloat16
        )
        pl.semaphore_wait(barrier_sem, 1)

        rdma0 = pltpu.make_async_remote_copy(
            src_ref=send_buf.at[0:R0],
            dst_ref=recv_buf.at[0:R0],
            send_sem=send_sems.at[0],
            recv_sem=recv_sems.at[0],
            device_id=peer,
            device_id_type=pl.DeviceIdType.MESH,
        )
        rdma0.start()

        send_buf[R0:M] = x_ref[0, R0:M, pl.ds(other_off, HALF)].astype(
            jnp.bfloat16
        )
        rdma1 = pltpu.make_async_remote_copy(
            src_ref=send_buf.at[R0:M],
            dst_ref=recv_buf.at[R0:M],
            send_sem=send_sems.at[1],
            recv_sem=recv_sems.at[1],
            device_id=peer,
            device_id_type=pl.DeviceIdType.MESH,
        )
        rdma1.start()

        my_off = my_z * HALF
        out_ref[...] = x_ref[0, :, pl.ds(my_off, HALF)]

        rdma0.wait()
        out_ref[0:R0] = out_ref[0:R0] + recv_buf[0:R0].astype(jnp.float32)
        rdma1.wait()
        out_ref[R0:M] = out_ref[R0:M] + recv_buf[R0:M].astype(jnp.float32)

    return pl.pallas_call(
        body,
        out_shape=jax.ShapeDtypeStruct((M, HALF), jnp.float32),
        in_specs=[pl.BlockSpec(memory_space=pltpu.VMEM)],
        out_specs=pl.BlockSpec(memory_space=pltpu.VMEM),
        scratch_shapes=[
            pltpu.VMEM((M, HALF), jnp.bfloat16),
            pltpu.VMEM((M, HALF), jnp.bfloat16),
            pltpu.SemaphoreType.DMA((2,)),
            pltpu.SemaphoreType.DMA((2,)),
        ],
        compiler_params=pltpu.CompilerParams(collective_id=0),
    )(x)


# device time: 11787 ns/iter; 1.0383x vs baseline; 1.0276x over previous
import jax
import jax.numpy as jnp
from jax import lax
from jax.experimental import pallas as pl
from jax.experimental.pallas import tpu as pltpu

M = 512
N = 1024
HALF = N // 2

F = 160
FC = F // 2
REST = M - 2 * F
DIRECT = F + REST


def kernel(x):
    def body(
        x_ref,
        out_ref,
        send_z,
        recv_z,
        recv_x,
        z_send_sems,
        z_recv_sems,
        x_send_sems,
        x_recv_sems,
    ):
        my_x = lax.axis_index("x")
        my_y = lax.axis_index("y")
        my_z = lax.axis_index("z")
        peer_z = (my_x, my_y, 1 - my_z)
        peer_x = (1 - my_x, my_y, my_z)

        barrier_sem = pltpu.get_barrier_semaphore()
        for nbr in (peer_z, peer_x):
            pl.semaphore_signal(
                barrier_sem,
                inc=1,
                device_id=nbr,
                device_id_type=pl.DeviceIdType.MESH,
            )

        other_off = (1 - my_z) * HALF
        fwd_row = my_x * F
        send_z[0:F] = x_ref[0, pl.ds(fwd_row, F), pl.ds(other_off, HALF)].astype(
            jnp.bfloat16
        )
        pl.semaphore_wait(barrier_sem, 2)

        def z_chunk(lo, hi, sem_idx):
            return pltpu.make_async_remote_copy(
                src_ref=send_z.at[lo:hi],
                dst_ref=recv_z.at[lo:hi],
                send_sem=z_send_sems.at[sem_idx],
                recv_sem=z_recv_sems.at[sem_idx],
                device_id=peer_z,
                device_id_type=pl.DeviceIdType.MESH,
            )

        za = z_chunk(0, FC, 0)
        za.start()
        zb = z_chunk(FC, F, 1)
        zb.start()

        send_z[F:DIRECT] = x_ref[
            0, pl.ds(2 * F, REST), pl.ds(other_off, HALF)
        ].astype(jnp.bfloat16)
        zc = z_chunk(F, DIRECT, 2)
        zc.start()

        my_off = my_z * HALF
        out_ref[...] = x_ref[0, :, pl.ds(my_off, HALF)]

        def x_fwd(lo, hi, sem_idx):
            return pltpu.make_async_remote_copy(
                src_ref=recv_z.at[lo:hi],
                dst_ref=recv_x.at[lo:hi],
                send_sem=x_send_sems.at[sem_idx],
                recv_sem=x_recv_sems.at[sem_idx],
                device_id=peer_x,
                device_id_type=pl.DeviceIdType.MESH,
            )

        za.wait()
        fa = x_fwd(0, FC, 0)
        fa.start()
        zb.wait()
        fb = x_fwd(FC, F, 1)
        fb.start()

        zc.wait()
        out_ref[pl.ds(fwd_row, F)] = out_ref[pl.ds(fwd_row, F)] + recv_z[
            0:F
        ].astype(jnp.float32)
        out_ref[2 * F : M] = out_ref[2 * F : M] + recv_z[F:DIRECT].astype(
            jnp.float32
        )

        fa.wait_send()
        fb.wait_send()
        rxa = pltpu.make_async_remote_copy(
            src_ref=recv_x.at[0:FC],
            dst_ref=recv_x.at[0:FC],
            send_sem=x_send_sems.at[0],
            recv_sem=x_recv_sems.at[0],
            device_id=peer_x,
            device_id_type=pl.DeviceIdType.MESH,
        )
        rxb = pltpu.make_async_remote_copy(
            src_ref=recv_x.at[FC:F],
            dst_ref=recv_x.at[FC:F],
            send_sem=x_send_sems.at[1],
            recv_sem=x_recv_sems.at[1],
            device_id=peer_x,
            device_id_type=pl.DeviceIdType.MESH,
        )
        rxa.wait_recv()
        rxb.wait_recv()
        nbr_row = (1 - my_x) * F
        out_ref[pl.ds(nbr_row, F)] = out_ref[pl.ds(nbr_row, F)] + recv_x[
            0:F
        ].astype(jnp.float32)

    return pl.pallas_call(
        body,
        out_shape=jax.ShapeDtypeStruct((M, HALF), jnp.float32),
        in_specs=[pl.BlockSpec(memory_space=pltpu.VMEM)],
        out_specs=pl.BlockSpec(memory_space=pltpu.VMEM),
        scratch_shapes=[
            pltpu.VMEM((DIRECT, HALF), jnp.bfloat16),
            pltpu.VMEM((DIRECT, HALF), jnp.bfloat16),
            pltpu.VMEM((F, HALF), jnp.bfloat16),
            pltpu.SemaphoreType.DMA((3,)),
            pltpu.SemaphoreType.DMA((3,)),
            pltpu.SemaphoreType.DMA((2,)),
            pltpu.SemaphoreType.DMA((2,)),
        ],
        compiler_params=pltpu.CompilerParams(collective_id=0),
    )(x)
